# baseline (speedup 1.0000x reference)
import functools

import jax
import jax.numpy as jnp
import numpy as np
from jax import lax
from jax.experimental import pallas as pl
from jax.experimental.pallas import tpu as pltpu
from jax.experimental.pallas import tpu_sc as plsc

MAX_PAD_LEN = 3072
MAX_SEQ_LEN = 2048
MIN_SEG_LEN = 19
MAX_SEG_LEN = 32
MAX_NUM_SEG = MAX_SEQ_LEN // MIN_SEG_LEN + 1
SEG_W = MAX_SEG_LEN * 2
B, T, D = 16, 2048, 512
M = MAX_NUM_SEG * SEG_W

NW = 32
C = 32
CHB = MAX_PAD_LEN // C
NCHUNKS = B * CHB
NCH = NCHUNKS // NW
NPAIR = NCH // 2 + 1
NV = D // 16
PACK_SHIFT = 13
JJP = 128


def _consts():
    def impl():
        bm = B * MAX_NUM_SEG
        key = jax.random.key(42)
        k_scale, k_len = jax.random.split(key)
        indices = jnp.broadcast_to(
            jnp.arange(SEG_W, dtype=jnp.float32)[None, :], (bm, SEG_W))
        scales = jax.random.uniform(k_scale, (bm,), dtype=jnp.float32) + 0.5
        idx_scaled = indices / scales[:, None]
        idx_scaled_fl = jnp.floor(idx_scaled)
        lambda_ = idx_scaled - idx_scaled_fl
        len_seg = jax.random.randint(
            k_len, (bm, 1), MIN_SEG_LEN, MAX_SEG_LEN, dtype=jnp.int32)
        offset = jnp.cumsum(len_seg.reshape(B, -1), axis=-1)
        offset = jnp.pad(offset[:, :-1], ((0, 0), (1, 0)))
        idx_scaled_org = idx_scaled_fl.reshape(B, MAX_NUM_SEG, SEG_W) + \
            offset.astype(jnp.float32)[:, :, None]
        i0 = jnp.clip(idx_scaled_org.astype(jnp.int32), 0, T - 2)
        g0f = (jnp.arange(B, dtype=jnp.int32)[:, None, None] * T
               + i0).astype(jnp.float32)
        meta = jnp.concatenate(
            [g0f.reshape(B * MAX_NUM_SEG, SEG_W),
             lambda_.reshape(B * MAX_NUM_SEG, SEG_W)],
            axis=-1)
        fl = idx_scaled_fl.reshape(B, MAX_NUM_SEG, SEG_W)
        len1 = (len_seg - 1).reshape(B, MAX_NUM_SEG)
        return fl, len1, offset, meta

    cpu = jax.local_devices(backend="cpu")[:1]
    cpu_mesh = jax.make_mesh((1,), ("_c",), devices=cpu)
    with jax.set_mesh(cpu_mesh):
        out = jax.jit(impl)()
        return tuple(np.asarray(o) for o in out)


_FL, _LEN1, _OFF, _META = _consts()


def _prep(seq_len):
    thr = jnp.minimum(
        jnp.asarray(_LEN1, jnp.float32),
        (seq_len[:, None] - 1 - jnp.asarray(_OFF)).astype(jnp.float32))
    v = jnp.sum(jnp.asarray(_FL) < thr[:, :, None], axis=-1,
                dtype=jnp.int32)
    cums = jnp.cumsum(v, axis=-1)
    seg_start = cums - v
    count = jnp.minimum(cums[:, -1], MAX_PAD_LEN)
    seg_ids = jnp.arange(MAX_NUM_SEG, dtype=jnp.int32)
    pack = (seg_ids << PACK_SHIFT) + seg_start
    p = jnp.arange(MAX_PAD_LEN, dtype=jnp.int32)
    le = seg_start[:, None, :] <= p[None, :, None]
    pmax = jnp.max(jnp.where(le, pack[:, None, :], 0), axis=-1)
    s_p = pmax >> PACK_SHIFT
    start_p = pmax & ((1 << PACK_SHIFT) - 1)
    jj = s_p * SEG_W + (p[None, :] - start_p)
    b_ix = jnp.arange(B, dtype=jnp.int32)[:, None]
    valid = p[None, :] < count[:, None]
    jj_enc = jnp.where(valid, b_ix * M + jj, -1)
    jj_r = jnp.pad(jj_enc.reshape(NCHUNKS, C),
                   ((0, 0), (0, JJP - C)))

    ngc = (count + C - 1) // C
    j = jnp.arange(CHB, dtype=jnp.int32)
    is_zero = (j[None, :] >= ngc[:, None]).reshape(-1)
    order = jnp.argsort(is_zero, stable=True).astype(jnp.int32)
    G = jnp.sum(ngc).astype(jnp.int32)
    slot_map = order.reshape(NCH, NW).T
    w_ids = jnp.arange(NW, dtype=jnp.int32)
    ng = jnp.maximum(0, (G - w_ids + NW - 1) // NW).astype(jnp.int32)
    return jj_r, ng, slot_map


def _splat0(ref, i):
    return plsc.load_gather(ref, [lax.broadcast(i, (16,))])[0]


def _sc_body(x_hbm, meta_hbm, jj_hbm, zz_hbm, ng_hbm, gc_hbm, out_hbm,
             jja_v, gc_v, ng_v, sg, meta, i0, i1, w0, w1, av, bv, ov,
             msem, xsem, ssem, semz):
    wid = lax.axis_index("s") * 2 + lax.axis_index("c")
    pltpu.sync_copy(ng_hbm, ng_v)
    pltpu.sync_copy(gc_hbm.at[wid], gc_v)
    myg = _splat0(ng_v, wid)
    lanes = lax.iota(jnp.int32, 16)

    pltpu.async_copy(jj_hbm.at[gc_v], jja_v, xsem[0]).wait()

    def build_sg(c, d):
        for k in range(C // 16):
            sl = pl.ds(k * 16, 16)
            sg[d][sl] = jnp.maximum(jja_v[c, sl], 0) >> 6

    def fire_meta(c, d):
        build_sg(c, d)
        pltpu.async_copy(meta_hbm.at[sg[d]], meta[d], msem[d])

    def consume_meta_fire_x(c, d):
        pltpu.make_async_copy(meta_hbm.at[pl.ds(0, C)], meta[d], msem[d]).wait()
        for k in range(C // 16):
            sl = pl.ds(k * 16, 16)
            je = jja_v[c, sl]
            jc = jnp.maximum(je, 0)
            row = lax.broadcast(jnp.int32(k * 16), (16,)) + lanes
            col = jc & (SEG_W - 1)
            g0f = plsc.load_gather(meta[d], [row, col])
            lam = plsc.load_gather(meta[d], [row, col + SEG_W])
            mf = jnp.where(je >= 0, 1.0, 0.0)
            g0i = g0f.astype(jnp.int32)
            i0[d][sl] = g0i
            i1[d][sl] = g0i + 1
            w1f = lam * mf
            w0[d][sl] = mf - w1f
            w1[d][sl] = w1f

        @pl.when(c + 2 < myg)
        def _():
            fire_meta(c + 2, d)

        pltpu.async_copy(x_hbm.at[i0[d]], av[d], xsem[d])
        pltpu.async_copy(x_hbm.at[i1[d]], bv[d], xsem[d])

    def compute_store(c, d):
        pltpu.make_async_copy(x_hbm.at[pl.ds(0, C)], av[d], xsem[d]).wait()
        pltpu.make_async_copy(x_hbm.at[pl.ds(0, C)], bv[d], xsem[d]).wait()

        @pl.when(c >= 2)
        def _():
            pltpu.make_async_copy(out_hbm.at[pl.ds(0, C)], ov[d], ssem[d]).wait()

        @pl.loop(0, C)
        def _row(r):
            w0s = plsc.load_gather(w0[d], [lax.broadcast(r, (16,))])
            w1s = plsc.load_gather(w1[d], [lax.broadcast(r, (16,))])
            for vv in range(NV):
                sl = pl.ds(vv * 16, 16)
                ov[d][r, sl] = w0s * av[d][r, sl] + w1s * bv[d][r, sl]

        base = pl.multiple_of(_splat0(gc_v, c) * C, C)
        pltpu.async_copy(ov[d], out_hbm.at[pl.ds(base, C)], ssem[d])

    @pl.when(myg >= 1)
    def _():
        fire_meta(0, 0)

    @pl.when(myg >= 2)
    def _():
        fire_meta(1, 1)

    @pl.loop(0, NPAIR)
    def _pair(i):
        e = 2 * i
        q = 2 * i + 1
        po = 2 * i - 1

        @pl.when(e < myg)
        def _():
            consume_meta_fire_x(e, 0)

        @pl.when((po >= 0) & (po < myg))
        def _():
            compute_store(po, 1)

        @pl.when(q < myg)
        def _():
            consume_meta_fire_x(q, 1)

        @pl.when(e < myg)
        def _():
            compute_store(e, 0)

        for zc in (myg + 2 * i, myg + 2 * i + 1):
            @pl.when(zc < NCH)
            def _(zc=zc):
                zb = pl.multiple_of(_splat0(gc_v, zc) * C, C)
                pltpu.async_copy(zz_hbm, out_hbm.at[pl.ds(zb, C)], semz)

    @pl.when(myg >= 1)
    def _():
        pltpu.make_async_copy(out_hbm.at[pl.ds(0, C)], ov[0], ssem[0]).wait()

    @pl.when(myg >= 2)
    def _():
        pltpu.make_async_copy(out_hbm.at[pl.ds(0, C)], ov[1], ssem[1]).wait()

    @pl.loop(myg, NCH)
    def _zdrain(c):
        pltpu.make_async_copy(zz_hbm, out_hbm.at[pl.ds(0, C)], semz).wait()


def kernel(x, seq_len):
    jj_r, ng, gc = _prep(seq_len)
    xf = x.reshape(B * T, D)
    meta = jnp.asarray(_META)

    mesh = plsc.VectorSubcoreMesh(core_axis_name="c", subcore_axis_name="s")
    run = functools.partial(
        pl.kernel,
        out_type=jax.ShapeDtypeStruct((B * MAX_PAD_LEN, D), jnp.float32),
        mesh=mesh,
        compiler_params=pltpu.CompilerParams(needs_layout_passes=False),
        scratch_types=[
            pltpu.VMEM((NCH, JJP), jnp.int32),
            pltpu.VMEM((NCH,), jnp.int32),
            pltpu.VMEM((NW,), jnp.int32),
            [pltpu.VMEM((C,), jnp.int32)] * 2,
            [pltpu.VMEM((C, 2 * SEG_W), jnp.float32)] * 2,
            [pltpu.VMEM((C,), jnp.int32)] * 2,
            [pltpu.VMEM((C,), jnp.int32)] * 2,
            [pltpu.VMEM((C,), jnp.float32)] * 2,
            [pltpu.VMEM((C,), jnp.float32)] * 2,
            [pltpu.VMEM((C, D), jnp.float32)] * 2,
            [pltpu.VMEM((C, D), jnp.float32)] * 2,
            [pltpu.VMEM((C, D), jnp.float32)] * 2,
            [pltpu.SemaphoreType.DMA] * 2,
            [pltpu.SemaphoreType.DMA] * 2,
            [pltpu.SemaphoreType.DMA] * 2,
            pltpu.SemaphoreType.DMA,
        ],
    )(_sc_body)
    zz = jnp.zeros((C, D), jnp.float32)
    out = run(xf, meta, jj_r, zz, ng, gc)
    return out.reshape(B, MAX_PAD_LEN, D)


# speedup vs baseline: 16.1561x; 16.1561x over previous
import functools

import jax
import jax.numpy as jnp
import numpy as np
from jax import lax
from jax.experimental import pallas as pl
from jax.experimental.pallas import tpu as pltpu
from jax.experimental.pallas import tpu_sc as plsc

MAX_PAD_LEN = 3072
MAX_SEQ_LEN = 2048
MIN_SEG_LEN = 19
MAX_SEG_LEN = 32
MAX_NUM_SEG = MAX_SEQ_LEN // MIN_SEG_LEN + 1
SEG_W = MAX_SEG_LEN * 2
B, T, D = 16, 2048, 512
M = MAX_NUM_SEG * SEG_W

NW = 32
C = 32
CHB = MAX_PAD_LEN // C
NCHUNKS = B * CHB
NCH = NCHUNKS // NW
NPAIR = NCH // 2 + 1
NV = D // 16
PACK_SHIFT = 13
JJP = 128


def _consts():
    def impl():
        bm = B * MAX_NUM_SEG
        key = jax.random.key(42)
        k_scale, k_len = jax.random.split(key)
        indices = jnp.broadcast_to(
            jnp.arange(SEG_W, dtype=jnp.float32)[None, :], (bm, SEG_W))
        scales = jax.random.uniform(k_scale, (bm,), dtype=jnp.float32) + 0.5
        idx_scaled = indices / scales[:, None]
        idx_scaled_fl = jnp.floor(idx_scaled)
        lambda_ = idx_scaled - idx_scaled_fl
        len_seg = jax.random.randint(
            k_len, (bm, 1), MIN_SEG_LEN, MAX_SEG_LEN, dtype=jnp.int32)
        offset = jnp.cumsum(len_seg.reshape(B, -1), axis=-1)
        offset = jnp.pad(offset[:, :-1], ((0, 0), (1, 0)))
        idx_scaled_org = idx_scaled_fl.reshape(B, MAX_NUM_SEG, SEG_W) + \
            offset.astype(jnp.float32)[:, :, None]
        i0 = jnp.clip(idx_scaled_org.astype(jnp.int32), 0, T - 2)
        g0f = (jnp.arange(B, dtype=jnp.int32)[:, None, None] * T
               + i0).astype(jnp.float32)
        meta = jnp.concatenate(
            [g0f.reshape(B * MAX_NUM_SEG, SEG_W),
             lambda_.reshape(B * MAX_NUM_SEG, SEG_W)],
            axis=-1)
        fl = idx_scaled_fl.reshape(B, MAX_NUM_SEG, SEG_W)
        len1 = (len_seg - 1).reshape(B, MAX_NUM_SEG)
        return fl, len1, offset, meta

    cpu = jax.local_devices(backend="cpu")[:1]
    cpu_mesh = jax.make_mesh((1,), ("_c",), devices=cpu)
    with jax.set_mesh(cpu_mesh):
        out = jax.jit(impl)()
        return tuple(np.asarray(o) for o in out)


_FL, _LEN1, _OFF, _META = _consts()


def _prep(seq_len):
    thr = jnp.minimum(
        jnp.asarray(_LEN1, jnp.float32),
        (seq_len[:, None] - 1 - jnp.asarray(_OFF)).astype(jnp.float32))
    v = jnp.sum(jnp.asarray(_FL) < thr[:, :, None], axis=-1,
                dtype=jnp.int32)
    cums = jnp.cumsum(v, axis=-1)
    seg_start = cums - v
    count = jnp.minimum(cums[:, -1], MAX_PAD_LEN)
    seg_ids = jnp.arange(MAX_NUM_SEG, dtype=jnp.int32)
    pack = (seg_ids << PACK_SHIFT) + seg_start
    p = jnp.arange(MAX_PAD_LEN, dtype=jnp.int32)
    le = seg_start[:, None, :] <= p[None, :, None]
    pmax = jnp.max(jnp.where(le, pack[:, None, :], 0), axis=-1)
    s_p = pmax >> PACK_SHIFT
    start_p = pmax & ((1 << PACK_SHIFT) - 1)
    jj = s_p * SEG_W + (p[None, :] - start_p)
    b_ix = jnp.arange(B, dtype=jnp.int32)[:, None]
    valid = p[None, :] < count[:, None]
    jj_enc = jnp.where(valid, b_ix * M + jj, -1)
    jj_r = jnp.pad(jj_enc.reshape(NCHUNKS, C),
                   ((0, 0), (0, JJP - C)))

    ngc = (count + C - 1) // C
    j = jnp.arange(CHB, dtype=jnp.int32)
    is_zero = (j[None, :] >= ngc[:, None]).reshape(-1)
    order = jnp.argsort(is_zero, stable=True).astype(jnp.int32)
    G = jnp.sum(ngc).astype(jnp.int32)
    slot_map = order.reshape(NCH, NW).T
    w_ids = jnp.arange(NW, dtype=jnp.int32)
    ng = jnp.maximum(0, (G - w_ids + NW - 1) // NW).astype(jnp.int32)
    return jj_r, ng, slot_map


def _splat0(ref, i):
    return plsc.load_gather(ref, [lax.broadcast(i, (16,))])[0]


def _sc_body(x_hbm, meta_hbm, jj_hbm, ng_hbm, gc_hbm, out_hbm,
             jja_v, gc_v, ng_v, sg, meta, i0, i1, w0, w1, av, bv, ov, z_v,
             msem, xsem, ssem, semz):
    wid = lax.axis_index("s") * 2 + lax.axis_index("c")
    pltpu.sync_copy(ng_hbm, ng_v)
    pltpu.sync_copy(gc_hbm.at[wid], gc_v)
    myg = _splat0(ng_v, wid)
    lanes = lax.iota(jnp.int32, 16)
    zv = jnp.zeros((16,), jnp.float32)

    @pl.loop(0, C)
    def _zfill(r):
        for vv in range(NV):
            z_v[r, pl.ds(vv * 16, 16)] = zv

    pltpu.async_copy(jj_hbm.at[gc_v], jja_v, xsem[0]).wait()

    def build_sg(c, d):
        for k in range(C // 16):
            sl = pl.ds(k * 16, 16)
            sg[d][sl] = jnp.maximum(jja_v[c, sl], 0) >> 6

    def fire_meta(c, d):
        build_sg(c, d)
        pltpu.async_copy(meta_hbm.at[sg[d]], meta[d], msem[d])

    def consume_meta_fire_x(c, d):
        pltpu.make_async_copy(meta_hbm.at[pl.ds(0, C)], meta[d], msem[d]).wait()
        for k in range(C // 16):
            sl = pl.ds(k * 16, 16)
            je = jja_v[c, sl]
            jc = jnp.maximum(je, 0)
            row = lax.broadcast(jnp.int32(k * 16), (16,)) + lanes
            col = jc & (SEG_W - 1)
            g0f = plsc.load_gather(meta[d], [row, col])
            lam = plsc.load_gather(meta[d], [row, col + SEG_W])
            mf = jnp.where(je >= 0, 1.0, 0.0)
            g0i = g0f.astype(jnp.int32)
            i0[d][sl] = g0i
            i1[d][sl] = g0i + 1
            w1f = lam * mf
            w0[d][sl] = mf - w1f
            w1[d][sl] = w1f

        @pl.when(c + 2 < myg)
        def _():
            fire_meta(c + 2, d)

        pltpu.async_copy(x_hbm.at[i0[d]], av[d], xsem[d])
        pltpu.async_copy(x_hbm.at[i1[d]], bv[d], xsem[d])

    def compute_store(c, d):
        pltpu.make_async_copy(x_hbm.at[pl.ds(0, C)], av[d], xsem[d]).wait()
        pltpu.make_async_copy(x_hbm.at[pl.ds(0, C)], bv[d], xsem[d]).wait()

        @pl.when(c >= 2)
        def _():
            pltpu.make_async_copy(out_hbm.at[pl.ds(0, C)], ov[d], ssem[d]).wait()

        @pl.loop(0, C)
        def _row(r):
            w0s = plsc.load_gather(w0[d], [lax.broadcast(r, (16,))])
            w1s = plsc.load_gather(w1[d], [lax.broadcast(r, (16,))])
            for vv in range(NV):
                sl = pl.ds(vv * 16, 16)
                ov[d][r, sl] = w0s * av[d][r, sl] + w1s * bv[d][r, sl]

        base = pl.multiple_of(_splat0(gc_v, c) * C, C)
        pltpu.async_copy(ov[d], out_hbm.at[pl.ds(base, C)], ssem[d])

    @pl.when(myg >= 1)
    def _():
        fire_meta(0, 0)

    @pl.when(myg >= 2)
    def _():
        fire_meta(1, 1)

    @pl.loop(0, NPAIR)
    def _pair(i):
        e = 2 * i
        q = 2 * i + 1
        po = 2 * i - 1

        @pl.when(e < myg)
        def _():
            consume_meta_fire_x(e, 0)

        @pl.when((po >= 0) & (po < myg))
        def _():
            compute_store(po, 1)

        @pl.when(q < myg)
        def _():
            consume_meta_fire_x(q, 1)

        @pl.when(e < myg)
        def _():
            compute_store(e, 0)

        for zc in (myg + 2 * i, myg + 2 * i + 1):
            @pl.when(zc < NCH)
            def _(zc=zc):
                zb = pl.multiple_of(_splat0(gc_v, zc) * C, C)
                pltpu.async_copy(z_v, out_hbm.at[pl.ds(zb, C)], semz)

    @pl.when(myg >= 1)
    def _():
        pltpu.make_async_copy(out_hbm.at[pl.ds(0, C)], ov[0], ssem[0]).wait()

    @pl.when(myg >= 2)
    def _():
        pltpu.make_async_copy(out_hbm.at[pl.ds(0, C)], ov[1], ssem[1]).wait()

    @pl.loop(myg, NCH)
    def _zdrain(c):
        pltpu.make_async_copy(out_hbm.at[pl.ds(0, C)], z_v, semz).wait()


def kernel(x, seq_len):
    jj_r, ng, gc = _prep(seq_len)
    xf = x.reshape(B * T, D)
    meta = jnp.asarray(_META)

    mesh = plsc.VectorSubcoreMesh(core_axis_name="c", subcore_axis_name="s")
    run = functools.partial(
        pl.kernel,
        out_type=jax.ShapeDtypeStruct((B * MAX_PAD_LEN, D), jnp.float32),
        mesh=mesh,
        compiler_params=pltpu.CompilerParams(needs_layout_passes=False),
        scratch_types=[
            pltpu.VMEM((NCH, JJP), jnp.int32),
            pltpu.VMEM((NCH,), jnp.int32),
            pltpu.VMEM((NW,), jnp.int32),
            [pltpu.VMEM((C,), jnp.int32)] * 2,
            [pltpu.VMEM((C, 2 * SEG_W), jnp.float32)] * 2,
            [pltpu.VMEM((C,), jnp.int32)] * 2,
            [pltpu.VMEM((C,), jnp.int32)] * 2,
            [pltpu.VMEM((C,), jnp.float32)] * 2,
            [pltpu.VMEM((C,), jnp.float32)] * 2,
            [pltpu.VMEM((C, D), jnp.float32)] * 2,
            [pltpu.VMEM((C, D), jnp.float32)] * 2,
            [pltpu.VMEM((C, D), jnp.float32)] * 2,
            pltpu.VMEM((C, D), jnp.float32),
            [pltpu.SemaphoreType.DMA] * 2,
            [pltpu.SemaphoreType.DMA] * 2,
            [pltpu.SemaphoreType.DMA] * 2,
            pltpu.SemaphoreType.DMA,
        ],
    )(_sc_body)
    out = run(xf, meta, jj_r, ng, gc)
    return out.reshape(B, MAX_PAD_LEN, D)


# speedup vs baseline: 16.3557x; 1.0124x over previous
import functools

import jax
import jax.numpy as jnp
import numpy as np
from jax import lax
from jax.experimental import pallas as pl
from jax.experimental.pallas import tpu as pltpu
from jax.experimental.pallas import tpu_sc as plsc

MAX_PAD_LEN = 3072
MAX_SEQ_LEN = 2048
MIN_SEG_LEN = 19
MAX_SEG_LEN = 32
MAX_NUM_SEG = MAX_SEQ_LEN // MIN_SEG_LEN + 1
SEG_W = MAX_SEG_LEN * 2
B, T, D = 16, 2048, 512
M = MAX_NUM_SEG * SEG_W

NW = 32
C = 32
CHB = MAX_PAD_LEN // C
NCHUNKS = B * CHB
NCH = NCHUNKS // NW
NPAIR = NCH // 2 + 1
NV = D // 16
PACK_SHIFT = 13
JJP = 128


def _consts():
    def impl():
        bm = B * MAX_NUM_SEG
        key = jax.random.key(42)
        k_scale, k_len = jax.random.split(key)
        indices = jnp.broadcast_to(
            jnp.arange(SEG_W, dtype=jnp.float32)[None, :], (bm, SEG_W))
        scales = jax.random.uniform(k_scale, (bm,), dtype=jnp.float32) + 0.5
        idx_scaled = indices / scales[:, None]
        idx_scaled_fl = jnp.floor(idx_scaled)
        lambda_ = idx_scaled - idx_scaled_fl
        len_seg = jax.random.randint(
            k_len, (bm, 1), MIN_SEG_LEN, MAX_SEG_LEN, dtype=jnp.int32)
        offset = jnp.cumsum(len_seg.reshape(B, -1), axis=-1)
        offset = jnp.pad(offset[:, :-1], ((0, 0), (1, 0)))
        idx_scaled_org = idx_scaled_fl.reshape(B, MAX_NUM_SEG, SEG_W) + \
            offset.astype(jnp.float32)[:, :, None]
        i0 = jnp.clip(idx_scaled_org.astype(jnp.int32), 0, T - 2)
        g0f = (jnp.arange(B, dtype=jnp.int32)[:, None, None] * T
               + i0).astype(jnp.float32)
        meta = jnp.concatenate(
            [g0f.reshape(B * MAX_NUM_SEG, SEG_W),
             lambda_.reshape(B * MAX_NUM_SEG, SEG_W)],
            axis=-1)
        fl = idx_scaled_fl.reshape(B, MAX_NUM_SEG, SEG_W)
        len1 = (len_seg - 1).reshape(B, MAX_NUM_SEG)
        return fl, len1, offset, meta

    cpu = jax.local_devices(backend="cpu")[:1]
    cpu_mesh = jax.make_mesh((1,), ("_c",), devices=cpu)
    with jax.set_mesh(cpu_mesh):
        out = jax.jit(impl)()
        return tuple(np.asarray(o) for o in out)


_FL, _LEN1, _OFF, _META = _consts()


def _prep(seq_len):
    thr = jnp.minimum(
        jnp.asarray(_LEN1, jnp.float32),
        (seq_len[:, None] - 1 - jnp.asarray(_OFF)).astype(jnp.float32))
    v = jnp.sum(jnp.asarray(_FL) < thr[:, :, None], axis=-1,
                dtype=jnp.int32)
    cums = jnp.cumsum(v, axis=-1)
    seg_start = cums - v
    count = jnp.minimum(cums[:, -1], MAX_PAD_LEN)
    seg_ids = jnp.arange(MAX_NUM_SEG, dtype=jnp.int32)
    pack = (seg_ids << PACK_SHIFT) + seg_start
    p = jnp.arange(MAX_PAD_LEN, dtype=jnp.int32)
    le = seg_start[:, None, :] <= p[None, :, None]
    pmax = jnp.max(jnp.where(le, pack[:, None, :], 0), axis=-1)
    s_p = pmax >> PACK_SHIFT
    start_p = pmax & ((1 << PACK_SHIFT) - 1)
    jj = s_p * SEG_W + (p[None, :] - start_p)
    b_ix = jnp.arange(B, dtype=jnp.int32)[:, None]
    valid = p[None, :] < count[:, None]
    jj_enc = jnp.where(valid, b_ix * M + jj, -1)
    jj_r = jnp.pad(jj_enc.reshape(NCHUNKS, C),
                   ((0, 0), (0, JJP - C)))

    ngc = (count + C - 1) // C
    j = jnp.arange(CHB, dtype=jnp.int32)
    is_zero = (j[None, :] >= ngc[:, None]).reshape(-1)
    order = jnp.argsort(is_zero, stable=True).astype(jnp.int32)
    G = jnp.sum(ngc).astype(jnp.int32)
    slot_map = order.reshape(NCH, NW).T
    w_ids = jnp.arange(NW, dtype=jnp.int32)
    ng = jnp.maximum(0, (G - w_ids + NW - 1) // NW).astype(jnp.int32)
    return jj_r, ng, slot_map


def _splat0(ref, i):
    return plsc.load_gather(ref, [lax.broadcast(i, (16,))])[0]


def _sc_body(x_hbm, meta_hbm, jj_hbm, ng_hbm, gc_hbm, out_hbm,
             jja_v, gc_v, ng_v, sg, meta, i0, i1, w0, w1, av, bv, ov, z_v,
             msem, xsem, ssem, semz):
    wid = lax.axis_index("s") * 2 + lax.axis_index("c")
    pltpu.sync_copy(ng_hbm, ng_v)
    pltpu.sync_copy(gc_hbm.at[wid], gc_v)
    myg = _splat0(ng_v, wid)
    lanes = lax.iota(jnp.int32, 16)
    zv = jnp.zeros((16,), jnp.float32)

    @pl.loop(0, C)
    def _zfill(r):
        for vv in range(NV):
            z_v[r, pl.ds(vv * 16, 16)] = zv

    pltpu.async_copy(jj_hbm.at[gc_v], jja_v, xsem[0]).wait()

    def build_sg(c, d):
        for k in range(C // 16):
            sl = pl.ds(k * 16, 16)
            sg[d][sl] = jnp.maximum(jja_v[c, sl], 0) >> 6

    def fire_meta(c, d):
        build_sg(c, d)
        pltpu.async_copy(meta_hbm.at[sg[d]], meta[d], msem[d])

    def consume_meta_fire_x(c, d):
        pltpu.make_async_copy(meta_hbm.at[pl.ds(0, C)], meta[d], msem[d]).wait()
        for k in range(C // 16):
            sl = pl.ds(k * 16, 16)
            je = jja_v[c, sl]
            jc = jnp.maximum(je, 0)
            row = lax.broadcast(jnp.int32(k * 16), (16,)) + lanes
            col = jc & (SEG_W - 1)
            g0f = plsc.load_gather(meta[d], [row, col])
            lam = plsc.load_gather(meta[d], [row, col + SEG_W])
            mf = jnp.where(je >= 0, 1.0, 0.0)
            g0i = g0f.astype(jnp.int32)
            i0[d][sl] = g0i
            i1[d][sl] = g0i + 1
            w1f = lam * mf
            w0[d][sl] = mf - w1f
            w1[d][sl] = w1f

        pltpu.async_copy(x_hbm.at[i0[d]], av[d], xsem[d])
        pltpu.async_copy(x_hbm.at[i1[d]], bv[d], xsem[d])

        @pl.when(c + 2 < myg)
        def _():
            fire_meta(c + 2, d)

    def compute_store(c, d):
        pltpu.make_async_copy(x_hbm.at[pl.ds(0, C)], av[d], xsem[d]).wait()
        pltpu.make_async_copy(x_hbm.at[pl.ds(0, C)], bv[d], xsem[d]).wait()

        @pl.when(c >= 2)
        def _():
            pltpu.make_async_copy(out_hbm.at[pl.ds(0, C)], ov[d], ssem[d]).wait()

        @pl.loop(0, C)
        def _row(r):
            w0s = plsc.load_gather(w0[d], [lax.broadcast(r, (16,))])
            w1s = plsc.load_gather(w1[d], [lax.broadcast(r, (16,))])
            for vv in range(NV):
                sl = pl.ds(vv * 16, 16)
                ov[d][r, sl] = w0s * av[d][r, sl] + w1s * bv[d][r, sl]

        base = pl.multiple_of(_splat0(gc_v, c) * C, C)
        pltpu.async_copy(ov[d], out_hbm.at[pl.ds(base, C)], ssem[d])

    @pl.when(myg >= 1)
    def _():
        fire_meta(0, 0)

    @pl.when(myg >= 2)
    def _():
        fire_meta(1, 1)

    @pl.loop(0, NPAIR)
    def _pair(i):
        e = 2 * i
        q = 2 * i + 1
        po = 2 * i - 1

        @pl.when(e < myg)
        def _():
            consume_meta_fire_x(e, 0)

        @pl.when((po >= 0) & (po < myg))
        def _():
            compute_store(po, 1)

        @pl.when(q < myg)
        def _():
            consume_meta_fire_x(q, 1)

        @pl.when(e < myg)
        def _():
            compute_store(e, 0)

        for zc in (myg + 2 * i, myg + 2 * i + 1):
            @pl.when(zc < NCH)
            def _(zc=zc):
                zb = pl.multiple_of(_splat0(gc_v, zc) * C, C)
                pltpu.async_copy(z_v, out_hbm.at[pl.ds(zb, C)], semz)

    @pl.when(myg >= 1)
    def _():
        pltpu.make_async_copy(out_hbm.at[pl.ds(0, C)], ov[0], ssem[0]).wait()

    @pl.when(myg >= 2)
    def _():
        pltpu.make_async_copy(out_hbm.at[pl.ds(0, C)], ov[1], ssem[1]).wait()

    @pl.loop(myg, NCH)
    def _zdrain(c):
        pltpu.make_async_copy(out_hbm.at[pl.ds(0, C)], z_v, semz).wait()


def kernel(x, seq_len):
    jj_r, ng, gc = _prep(seq_len)
    xf = x.reshape(B * T, D)
    meta = jnp.asarray(_META)

    mesh = plsc.VectorSubcoreMesh(core_axis_name="c", subcore_axis_name="s")
    run = functools.partial(
        pl.kernel,
        out_type=jax.ShapeDtypeStruct((B * MAX_PAD_LEN, D), jnp.float32),
        mesh=mesh,
        compiler_params=pltpu.CompilerParams(needs_layout_passes=False),
        scratch_types=[
            pltpu.VMEM((NCH, JJP), jnp.int32),
            pltpu.VMEM((NCH,), jnp.int32),
            pltpu.VMEM((NW,), jnp.int32),
            [pltpu.VMEM((C,), jnp.int32)] * 2,
            [pltpu.VMEM((C, 2 * SEG_W), jnp.float32)] * 2,
            [pltpu.VMEM((C,), jnp.int32)] * 2,
            [pltpu.VMEM((C,), jnp.int32)] * 2,
            [pltpu.VMEM((C,), jnp.float32)] * 2,
            [pltpu.VMEM((C,), jnp.float32)] * 2,
            [pltpu.VMEM((C, D), jnp.float32)] * 2,
            [pltpu.VMEM((C, D), jnp.float32)] * 2,
            [pltpu.VMEM((C, D), jnp.float32)] * 2,
            pltpu.VMEM((C, D), jnp.float32),
            [pltpu.SemaphoreType.DMA] * 2,
            [pltpu.SemaphoreType.DMA] * 2,
            [pltpu.SemaphoreType.DMA] * 2,
            pltpu.SemaphoreType.DMA,
        ],
    )(_sc_body)
    out = run(xf, meta, jj_r, ng, gc)
    return out.reshape(B, MAX_PAD_LEN, D)
